# baseline (speedup 1.0000x reference)
import jax
import jax.numpy as jnp
from jax import lax
from jax.experimental import pallas as pl
from jax.experimental.pallas import tpu as pltpu
from jax.experimental.pallas import tpu_sc as plsc

DIM = 64
PACK = 128
WPR = DIM // 2
BATCH = 16384
CTX = 20

_NC = 2
_NS = 16
_NW = _NC * _NS
_BPW = BATCH // _NW
_GB = 16
_NG = _BPW // _GB
_ROWS = _GB * CTX
_HALF = CTX // 2


def _sc_body(t_hbm, c_hbm, skip_hbm, ctxtab_hbm, out_hbm,
             tidx, thv, cidx, chv, tgt_v, ctx_v, out_v, sem):
    wid = lax.axis_index("s") * _NC + lax.axis_index("c")
    iota = lax.broadcasted_iota(jnp.int32, (16,), 0)
    iota_ctx = iota * CTX

    def group(g, carry):
        b0 = wid * _BPW + g * _GB
        p0 = b0 * CTX
        pltpu.sync_copy(t_hbm.at[pl.ds(b0, _GB)], tidx)
        pltpu.sync_copy(c_hbm.at[pl.ds(p0, _ROWS)], cidx)
        tid = tidx[...]
        thv[...] = tid >> 2
        for k in range(CTX):
            chv[pl.ds(k * _GB, _GB)] = cidx[pl.ds(k * _GB, _GB)] >> 2
        cps = [pltpu.async_copy(skip_hbm.at[thv], tgt_v, sem),
               pltpu.async_copy(ctxtab_hbm.at[chv], ctx_v, sem)]
        for cp in cps:
            cp.wait()
        tb32 = (tid & 3) << 5

        for h in range(2):
            cs = range(h * _HALF, (h + 1) * _HALF)
            cb32 = [(plsc.load_gather(cidx, [iota_ctx + c]) & 3) << 5
                    for c in cs]

            def wstep(w, accs):
                tw = plsc.load_gather(tgt_v, [iota, tb32 + w])
                ta, tb = plsc.unpack(plsc.bitcast(tw, jnp.bfloat16),
                                     format=plsc.PackFormat.INTERLEAVED)
                out = []
                for j, c in enumerate(cs):
                    cw = plsc.load_gather(ctx_v, [iota_ctx + c, cb32[j] + w])
                    ca, cb = plsc.unpack(plsc.bitcast(cw, jnp.bfloat16),
                                         format=plsc.PackFormat.INTERLEAVED)
                    out.append(accs[j] + ta * ca + tb * cb)
                return tuple(out)

            accs = lax.fori_loop(
                0, WPR, wstep,
                tuple(jnp.zeros((16,), jnp.float32) for _ in cs))
            for j, c in enumerate(cs):
                plsc.store_scatter(out_v, [iota_ctx + c], accs[j])
        pltpu.sync_copy(out_v, out_hbm.at[pl.ds(p0, _ROWS)])
        return carry

    lax.fori_loop(0, _NG, group, 0)


def _pack_table(tab):
    tb = tab.astype(jnp.bfloat16).reshape(-1, WPR, 2)
    return jax.lax.bitcast_convert_type(tb, jnp.int32).reshape(-1, PACK)


def kernel(target, context, skipgram_table, context_table):
    mesh = plsc.VectorSubcoreMesh(core_axis_name="c", subcore_axis_name="s")
    f = pl.kernel(
        _sc_body,
        out_type=jax.ShapeDtypeStruct((BATCH * CTX,), jnp.float32),
        mesh=mesh,
        scratch_types=[
            pltpu.VMEM((_GB,), jnp.int32),
            pltpu.VMEM((_GB,), jnp.int32),
            pltpu.VMEM((_ROWS,), jnp.int32),
            pltpu.VMEM((_ROWS,), jnp.int32),
            pltpu.VMEM((_GB, PACK), jnp.int32),
            pltpu.VMEM((_ROWS, PACK), jnp.int32),
            pltpu.VMEM((_ROWS,), jnp.float32),
            pltpu.SemaphoreType.DMA,
        ],
        compiler_params=pltpu.CompilerParams(
            needs_layout_passes=False, use_tc_tiling_on_sc=True),
    )
    out = f(target.astype(jnp.int32), context.reshape(-1).astype(jnp.int32),
            _pack_table(skipgram_table), _pack_table(context_table))
    return out.reshape(BATCH, CTX)


# speedup vs baseline: 2.7468x; 2.7468x over previous
import jax
import jax.numpy as jnp
from jax import lax
from jax.experimental import pallas as pl
from jax.experimental.pallas import tpu as pltpu
from jax.experimental.pallas import tpu_sc as plsc

DIM = 64
PACK = 128
WPR = DIM // 2
BATCH = 16384
CTX = 20

_NC = 2
_NS = 16
_NW = _NC * _NS
_BPW = BATCH // _NW
_GB = 16
_NG = _BPW // _GB
_ROWS = _GB * CTX
_HALF = CTX // 2


def _sc_body(t_hbm, c_hbm, skip_hbm, ctxtab_hbm, out_hbm,
             tidx, thv, cidx, chv, tgt_v, ctx_v, out_v, sem):
    wid = lax.axis_index("s") * _NC + lax.axis_index("c")
    iota = lax.broadcasted_iota(jnp.int32, (16,), 0)
    iota_ctx = iota * CTX

    def group(g, carry):
        b0 = wid * _BPW + g * _GB
        p0 = b0 * CTX
        pltpu.sync_copy(t_hbm.at[pl.ds(b0, _GB)], tidx)
        pltpu.sync_copy(c_hbm.at[pl.ds(p0, _ROWS)], cidx)
        tid = tidx[...]
        thv[...] = ((tid >> 11) << 9) | (tid & 511)
        for k in range(CTX):
            cv = cidx[pl.ds(k * _GB, _GB)]
            chv[pl.ds(k * _GB, _GB)] = ((cv >> 11) << 9) | (cv & 511)
        cps = [pltpu.async_copy(skip_hbm.at[thv], tgt_v, sem),
               pltpu.async_copy(ctxtab_hbm.at[chv], ctx_v, sem)]
        for cp in cps:
            cp.wait()
        tb32 = ((tid >> 9) & 3) << 5

        for h in range(2):
            cs = range(h * _HALF, (h + 1) * _HALF)
            cb32 = [((plsc.load_gather(cidx, [iota_ctx + c]) >> 9) & 3) << 5
                    for c in cs]

            def wstep(w, accs):
                tw = plsc.load_gather(tgt_v, [iota, tb32 + w])
                ta, tb = plsc.unpack(plsc.bitcast(tw, jnp.bfloat16),
                                     format=plsc.PackFormat.INTERLEAVED)
                out = []
                for j, c in enumerate(cs):
                    cw = plsc.load_gather(ctx_v, [iota_ctx + c, cb32[j] + w])
                    ca, cb = plsc.unpack(plsc.bitcast(cw, jnp.bfloat16),
                                         format=plsc.PackFormat.INTERLEAVED)
                    out.append(accs[j] + ta * ca + tb * cb)
                return tuple(out)

            accs = lax.fori_loop(
                0, WPR, wstep,
                tuple(jnp.zeros((16,), jnp.float32) for _ in cs))
            for j, c in enumerate(cs):
                plsc.store_scatter(out_v, [iota_ctx + c], accs[j])
        pltpu.sync_copy(out_v, out_hbm.at[pl.ds(p0, _ROWS)])
        return carry

    lax.fori_loop(0, _NG, group, 0)


_VCH = 2048
_VOCAB = 1000000
_TGRID = -(-_VOCAB // _VCH)


def _pack_body(tt_ref, out_ref):
    x = tt_ref[...]
    y = jax.lax.bitcast_convert_type(x, jnp.int32)
    rne = y + jnp.int32(0x7FFF) + (lax.shift_right_logical(y, 16) & 1)
    bf = lax.shift_right_logical(rne, 16)
    bf3 = bf.reshape(WPR, 2, _VCH)
    lo = bf3[:, 0, :]
    hi = bf3[:, 1, :]
    w = lo | (hi << 16)
    wt = w.T
    q = _VCH // 4
    out_ref[...] = jnp.concatenate(
        [wt[a * q:(a + 1) * q, :] for a in range(4)], axis=1)


def _pack_table(tab):
    f = pl.pallas_call(
        _pack_body,
        grid=(_TGRID,),
        in_specs=[pl.BlockSpec((DIM, _VCH), lambda k: (0, k))],
        out_specs=pl.BlockSpec((_VCH // 4, PACK), lambda k: (k, 0)),
        out_shape=jax.ShapeDtypeStruct((_TGRID * (_VCH // 4), PACK), jnp.int32),
    )
    return f(tab.T)


def kernel(target, context, skipgram_table, context_table):
    mesh = plsc.VectorSubcoreMesh(core_axis_name="c", subcore_axis_name="s")
    f = pl.kernel(
        _sc_body,
        out_type=jax.ShapeDtypeStruct((BATCH * CTX,), jnp.float32),
        mesh=mesh,
        scratch_types=[
            pltpu.VMEM((_GB,), jnp.int32),
            pltpu.VMEM((_GB,), jnp.int32),
            pltpu.VMEM((_ROWS,), jnp.int32),
            pltpu.VMEM((_ROWS,), jnp.int32),
            pltpu.VMEM((_GB, PACK), jnp.int32),
            pltpu.VMEM((_ROWS, PACK), jnp.int32),
            pltpu.VMEM((_ROWS,), jnp.float32),
            pltpu.SemaphoreType.DMA,
        ],
        compiler_params=pltpu.CompilerParams(
            needs_layout_passes=False, use_tc_tiling_on_sc=True),
    )
    out = f(target.astype(jnp.int32), context.reshape(-1).astype(jnp.int32),
            _pack_table(skipgram_table), _pack_table(context_table))
    return out.reshape(BATCH, CTX)


# speedup vs baseline: 2.8595x; 1.0410x over previous
import jax
import jax.numpy as jnp
from jax import lax
from jax.experimental import pallas as pl
from jax.experimental.pallas import tpu as pltpu
from jax.experimental.pallas import tpu_sc as plsc

DIM = 64
PACK = 128
WPR = DIM // 2
BATCH = 16384
CTX = 20

_NC = 2
_NS = 16
_NW = _NC * _NS
_BPW = BATCH // _NW
_GB = 16
_NG = _BPW // _GB
_ROWS = _GB * CTX
_HALF = CTX // 2


def _sc_body(t_hbm, c_hbm, skip_hbm, ctxtab_hbm, out_hbm,
             tidx, thv, cidx, chv, tgt_v, ctx_v, out_v, sem):
    wid = lax.axis_index("s") * _NC + lax.axis_index("c")
    iota = lax.broadcasted_iota(jnp.int32, (16,), 0)
    iota_ctx = iota * CTX

    def group(g, carry):
        b0 = wid * _BPW + g * _GB
        p0 = b0 * CTX
        pltpu.sync_copy(t_hbm.at[pl.ds(b0, _GB)], tidx)
        pltpu.sync_copy(c_hbm.at[pl.ds(p0, _ROWS)], cidx)
        tid = tidx[...]
        thv[...] = ((tid >> 11) << 9) | (tid & 511)
        for k in range(CTX):
            cv = cidx[pl.ds(k * _GB, _GB)]
            chv[pl.ds(k * _GB, _GB)] = ((cv >> 11) << 9) | (cv & 511)
        cps = [pltpu.async_copy(skip_hbm.at[thv], tgt_v, sem),
               pltpu.async_copy(ctxtab_hbm.at[chv], ctx_v, sem)]
        for cp in cps:
            cp.wait()
        tb32 = ((tid >> 9) & 3) << 5

        for h in range(2):
            cs = range(h * _HALF, (h + 1) * _HALF)
            cb32 = [((plsc.load_gather(cidx, [iota_ctx + c]) >> 9) & 3) << 5
                    for c in cs]

            def wstep(w, accs):
                tw = plsc.load_gather(tgt_v, [iota, tb32 + w])
                ta, tb = plsc.unpack(plsc.bitcast(tw, jnp.bfloat16),
                                     format=plsc.PackFormat.INTERLEAVED)
                out = []
                for j, c in enumerate(cs):
                    cw = plsc.load_gather(ctx_v, [iota_ctx + c, cb32[j] + w])
                    ca, cb = plsc.unpack(plsc.bitcast(cw, jnp.bfloat16),
                                         format=plsc.PackFormat.INTERLEAVED)
                    out.append(accs[j] + ta * ca + tb * cb)
                return tuple(out)

            accs = lax.fori_loop(
                0, WPR, wstep,
                tuple(jnp.zeros((16,), jnp.float32) for _ in cs))
            for j, c in enumerate(cs):
                plsc.store_scatter(out_v, [iota_ctx + c], accs[j])
        pltpu.sync_copy(out_v, out_hbm.at[pl.ds(p0, _ROWS)])
        return carry

    lax.fori_loop(0, _NG, group, 0)


_VCH = 2048
_VOCAB = 1000000
_TGRID = -(-_VOCAB // _VCH)


def _pack_body(tt_ref, out_ref):
    xb = tt_ref[...].astype(jnp.bfloat16)
    r = lax.broadcasted_iota(jnp.int32, (DIM, WPR), 0)
    c = lax.broadcasted_iota(jnp.int32, (DIM, WPR), 1)
    ev = (r == 2 * c).astype(jnp.bfloat16)
    od = (r == 2 * c + 1).astype(jnp.bfloat16)
    dn = (((0,), (0,)), ((), ()))
    lo = lax.dot_general(xb, ev, dn, preferred_element_type=jnp.float32)
    hi = lax.dot_general(xb, od, dn, preferred_element_type=jnp.float32)
    li = jax.lax.bitcast_convert_type(lo, jnp.int32)
    hj = jax.lax.bitcast_convert_type(hi, jnp.int32)
    w = (hj & jnp.int32(-65536)) | lax.shift_right_logical(li, 16)
    q = _VCH // 4
    out_ref[...] = jnp.concatenate(
        [w[a * q:(a + 1) * q, :] for a in range(4)], axis=1)


def _pack_table(tab):
    f = pl.pallas_call(
        _pack_body,
        grid=(_TGRID,),
        in_specs=[pl.BlockSpec((DIM, _VCH), lambda k: (0, k))],
        out_specs=pl.BlockSpec((_VCH // 4, PACK), lambda k: (k, 0)),
        out_shape=jax.ShapeDtypeStruct((_TGRID * (_VCH // 4), PACK), jnp.int32),
    )
    return f(tab.T)


def kernel(target, context, skipgram_table, context_table):
    mesh = plsc.VectorSubcoreMesh(core_axis_name="c", subcore_axis_name="s")
    f = pl.kernel(
        _sc_body,
        out_type=jax.ShapeDtypeStruct((BATCH * CTX,), jnp.float32),
        mesh=mesh,
        scratch_types=[
            pltpu.VMEM((_GB,), jnp.int32),
            pltpu.VMEM((_GB,), jnp.int32),
            pltpu.VMEM((_ROWS,), jnp.int32),
            pltpu.VMEM((_ROWS,), jnp.int32),
            pltpu.VMEM((_GB, PACK), jnp.int32),
            pltpu.VMEM((_ROWS, PACK), jnp.int32),
            pltpu.VMEM((_ROWS,), jnp.float32),
            pltpu.SemaphoreType.DMA,
        ],
        compiler_params=pltpu.CompilerParams(
            needs_layout_passes=False, use_tc_tiling_on_sc=True),
    )
    out = f(target.astype(jnp.int32), context.reshape(-1).astype(jnp.int32),
            _pack_table(skipgram_table), _pack_table(context_table))
    return out.reshape(BATCH, CTX)


# speedup vs baseline: 4.7616x; 1.6652x over previous
import jax
import jax.numpy as jnp
from jax import lax
from jax.experimental import pallas as pl
from jax.experimental.pallas import tpu as pltpu
from jax.experimental.pallas import tpu_sc as plsc

DIM = 64
PACK = 128
WPR = DIM // 2
BATCH = 16384
CTX = 20

_NC = 2
_NS = 16
_NW = _NC * _NS
_BPW = BATCH // _NW
_GB = 16
_NG = _BPW // _GB
_ROWS = _GB * CTX
_HALF = CTX // 2


def _sc_body(t_hbm, c_hbm, skip_hbm, ctxtab_hbm, out_hbm,
             tidx, thv, cidx, chv, tgt_v, ctx_v, out_v, sem):
    wid = lax.axis_index("s") * _NC + lax.axis_index("c")
    iota = lax.broadcasted_iota(jnp.int32, (16,), 0)
    iota_ctx = iota * CTX

    def group(g, carry):
        b0 = wid * _BPW + g * _GB
        p0 = b0 * CTX
        pltpu.sync_copy(t_hbm.at[pl.ds(b0, _GB)], tidx)
        pltpu.sync_copy(c_hbm.at[pl.ds(p0, _ROWS)], cidx)
        tid = tidx[...]
        thv[...] = ((tid >> 14) << 12) | (tid & 4095)
        for k in range(CTX):
            cv = cidx[pl.ds(k * _GB, _GB)]
            chv[pl.ds(k * _GB, _GB)] = ((cv >> 14) << 12) | (cv & 4095)
        cps = [pltpu.async_copy(skip_hbm.at[thv], tgt_v, sem),
               pltpu.async_copy(ctxtab_hbm.at[chv], ctx_v, sem)]
        for cp in cps:
            cp.wait()
        tb32 = ((tid >> 12) & 3) << 5

        for h in range(2):
            cs = range(h * _HALF, (h + 1) * _HALF)
            cb32 = [((plsc.load_gather(cidx, [iota_ctx + c]) >> 12) & 3) << 5
                    for c in cs]

            def wstep(w, accs):
                tw = plsc.load_gather(tgt_v, [iota, tb32 + w])
                ta, tb = plsc.unpack(plsc.bitcast(tw, jnp.bfloat16),
                                     format=plsc.PackFormat.INTERLEAVED)
                out = []
                for j, c in enumerate(cs):
                    cw = plsc.load_gather(ctx_v, [iota_ctx + c, cb32[j] + w])
                    ca, cb = plsc.unpack(plsc.bitcast(cw, jnp.bfloat16),
                                         format=plsc.PackFormat.INTERLEAVED)
                    out.append(accs[j] + ta * ca + tb * cb)
                return tuple(out)

            accs = lax.fori_loop(
                0, WPR, wstep,
                tuple(jnp.zeros((16,), jnp.float32) for _ in cs))
            for j, c in enumerate(cs):
                plsc.store_scatter(out_v, [iota_ctx + c], accs[j])
        pltpu.sync_copy(out_v, out_hbm.at[pl.ds(p0, _ROWS)])
        return carry

    lax.fori_loop(0, _NG, group, 0)


_VCH = 16384
_VOCAB = 1000000
_TGRID = -(-_VOCAB // _VCH)


def _pack_body(tt_ref, out_ref):
    xb = tt_ref[...].astype(jnp.bfloat16)
    r = lax.broadcasted_iota(jnp.int32, (DIM, WPR), 0)
    c = lax.broadcasted_iota(jnp.int32, (DIM, WPR), 1)
    ev = (r == 2 * c).astype(jnp.bfloat16)
    od = (r == 2 * c + 1).astype(jnp.bfloat16)
    dn = (((0,), (0,)), ((), ()))
    lo = lax.dot_general(xb, ev, dn, preferred_element_type=jnp.float32)
    hi = lax.dot_general(xb, od, dn, preferred_element_type=jnp.float32)
    li = jax.lax.bitcast_convert_type(lo, jnp.int32)
    hj = jax.lax.bitcast_convert_type(hi, jnp.int32)
    w = (hj & jnp.int32(-65536)) | lax.shift_right_logical(li, 16)
    q = _VCH // 4
    out_ref[...] = jnp.concatenate(
        [w[a * q:(a + 1) * q, :] for a in range(4)], axis=1)


def _pack_table(tab):
    f = pl.pallas_call(
        _pack_body,
        grid=(_TGRID,),
        in_specs=[pl.BlockSpec((DIM, _VCH), lambda k: (0, k))],
        out_specs=pl.BlockSpec((_VCH // 4, PACK), lambda k: (k, 0)),
        out_shape=jax.ShapeDtypeStruct((_TGRID * (_VCH // 4), PACK), jnp.int32),
    )
    return f(tab.T)


def kernel(target, context, skipgram_table, context_table):
    mesh = plsc.VectorSubcoreMesh(core_axis_name="c", subcore_axis_name="s")
    f = pl.kernel(
        _sc_body,
        out_type=jax.ShapeDtypeStruct((BATCH * CTX,), jnp.float32),
        mesh=mesh,
        scratch_types=[
            pltpu.VMEM((_GB,), jnp.int32),
            pltpu.VMEM((_GB,), jnp.int32),
            pltpu.VMEM((_ROWS,), jnp.int32),
            pltpu.VMEM((_ROWS,), jnp.int32),
            pltpu.VMEM((_GB, PACK), jnp.int32),
            pltpu.VMEM((_ROWS, PACK), jnp.int32),
            pltpu.VMEM((_ROWS,), jnp.float32),
            pltpu.SemaphoreType.DMA,
        ],
        compiler_params=pltpu.CompilerParams(
            needs_layout_passes=False, use_tc_tiling_on_sc=True),
    )
    out = f(target.astype(jnp.int32), context.reshape(-1).astype(jnp.int32),
            _pack_table(skipgram_table), _pack_table(context_table))
    return out.reshape(BATCH, CTX)


# speedup vs baseline: 4.8672x; 1.0222x over previous
import jax
import jax.numpy as jnp
from jax import lax
from jax.experimental import pallas as pl
from jax.experimental.pallas import tpu as pltpu
from jax.experimental.pallas import tpu_sc as plsc

DIM = 64
PACK = 128
WPR = DIM // 2
BATCH = 16384
CTX = 20

_NC = 2
_NS = 16
_NW = _NC * _NS
_BPW = BATCH // _NW
_GB = 16
_NG = _BPW // _GB
_ROWS = _GB * CTX
_HALF = CTX // 2


def _sc_body(t_hbm, c_hbm, skip_hbm, ctxtab_hbm, out_hbm,
             tidx, thv, cidx, chv, tgt_v, ctx_v, out_v, sem):
    wid = lax.axis_index("s") * _NC + lax.axis_index("c")
    iota = lax.broadcasted_iota(jnp.int32, (16,), 0)
    iota_ctx = iota * CTX

    def group(g, carry):
        b0 = wid * _BPW + g * _GB
        p0 = b0 * CTX
        pltpu.sync_copy(t_hbm.at[pl.ds(b0, _GB)], tidx)
        pltpu.sync_copy(c_hbm.at[pl.ds(p0, _ROWS)], cidx)
        tid = tidx[...]
        thv[...] = ((tid >> 15) << 13) | (tid & 8191)
        for k in range(CTX):
            cv = cidx[pl.ds(k * _GB, _GB)]
            chv[pl.ds(k * _GB, _GB)] = ((cv >> 15) << 13) | (cv & 8191)
        cps = [pltpu.async_copy(skip_hbm.at[thv], tgt_v, sem),
               pltpu.async_copy(ctxtab_hbm.at[chv], ctx_v, sem)]
        for cp in cps:
            cp.wait()
        tb32 = ((tid >> 13) & 3) << 5

        for h in range(2):
            cs = range(h * _HALF, (h + 1) * _HALF)
            cb32 = [((plsc.load_gather(cidx, [iota_ctx + c]) >> 13) & 3) << 5
                    for c in cs]

            def wstep(w, accs):
                tw = plsc.load_gather(tgt_v, [iota, tb32 + w])
                ta, tb = plsc.unpack(plsc.bitcast(tw, jnp.bfloat16),
                                     format=plsc.PackFormat.INTERLEAVED)
                out = []
                for j, c in enumerate(cs):
                    cw = plsc.load_gather(ctx_v, [iota_ctx + c, cb32[j] + w])
                    ca, cb = plsc.unpack(plsc.bitcast(cw, jnp.bfloat16),
                                         format=plsc.PackFormat.INTERLEAVED)
                    out.append(accs[j] + ta * ca + tb * cb)
                return tuple(out)

            accs = lax.fori_loop(
                0, WPR, wstep,
                tuple(jnp.zeros((16,), jnp.float32) for _ in cs))
            for j, c in enumerate(cs):
                plsc.store_scatter(out_v, [iota_ctx + c], accs[j])
        pltpu.sync_copy(out_v, out_hbm.at[pl.ds(p0, _ROWS)])
        return carry

    lax.fori_loop(0, _NG, group, 0)


_VCH = 32768
_VOCAB = 1000000
_TGRID = -(-_VOCAB // _VCH)


def _pack_body(tt_ref, out_ref):
    xb = tt_ref[...].astype(jnp.bfloat16)
    r = lax.broadcasted_iota(jnp.int32, (DIM, WPR), 0)
    c = lax.broadcasted_iota(jnp.int32, (DIM, WPR), 1)
    ev = (r == 2 * c).astype(jnp.bfloat16)
    od = (r == 2 * c + 1).astype(jnp.bfloat16)
    dn = (((0,), (0,)), ((), ()))
    lo = lax.dot_general(xb, ev, dn, preferred_element_type=jnp.float32)
    hi = lax.dot_general(xb, od, dn, preferred_element_type=jnp.float32)
    li = jax.lax.bitcast_convert_type(lo, jnp.int32)
    hj = jax.lax.bitcast_convert_type(hi, jnp.int32)
    w = (hj & jnp.int32(-65536)) | lax.shift_right_logical(li, 16)
    q = _VCH // 4
    out_ref[...] = jnp.concatenate(
        [w[a * q:(a + 1) * q, :] for a in range(4)], axis=1)


def _pack_table(tab):
    f = pl.pallas_call(
        _pack_body,
        grid=(_TGRID,),
        in_specs=[pl.BlockSpec((DIM, _VCH), lambda k: (0, k))],
        out_specs=pl.BlockSpec((_VCH // 4, PACK), lambda k: (k, 0)),
        out_shape=jax.ShapeDtypeStruct((_TGRID * (_VCH // 4), PACK), jnp.int32),
    )
    return f(tab.T)


def kernel(target, context, skipgram_table, context_table):
    mesh = plsc.VectorSubcoreMesh(core_axis_name="c", subcore_axis_name="s")
    f = pl.kernel(
        _sc_body,
        out_type=jax.ShapeDtypeStruct((BATCH * CTX,), jnp.float32),
        mesh=mesh,
        scratch_types=[
            pltpu.VMEM((_GB,), jnp.int32),
            pltpu.VMEM((_GB,), jnp.int32),
            pltpu.VMEM((_ROWS,), jnp.int32),
            pltpu.VMEM((_ROWS,), jnp.int32),
            pltpu.VMEM((_GB, PACK), jnp.int32),
            pltpu.VMEM((_ROWS, PACK), jnp.int32),
            pltpu.VMEM((_ROWS,), jnp.float32),
            pltpu.SemaphoreType.DMA,
        ],
        compiler_params=pltpu.CompilerParams(
            needs_layout_passes=False, use_tc_tiling_on_sc=True),
    )
    out = f(target.astype(jnp.int32), context.reshape(-1).astype(jnp.int32),
            _pack_table(skipgram_table), _pack_table(context_table))
    return out.reshape(BATCH, CTX)


# speedup vs baseline: 5.6333x; 1.1574x over previous
import jax
import jax.numpy as jnp
from jax import lax
from jax.experimental import pallas as pl
from jax.experimental.pallas import tpu as pltpu
from jax.experimental.pallas import tpu_sc as plsc

DIM = 64
PACK = 128
WPR = DIM // 2
BATCH = 16384
CTX = 20

_NC = 2
_NS = 16
_NW = _NC * _NS
_BPW = BATCH // _NW
_GB = 16
_NG = _BPW // _GB
_ROWS = _GB * CTX
_HALF = CTX // 2

_VCH = 32768
_VSH = 15
_QSH = _VSH - 2
_QM = (1 << _QSH) - 1
_VOCAB = 1000000
_TGRID = -(-_VOCAB // _VCH)


def _sc_body(t_hbm, c_hbm, skip_hbm, ctxtab_hbm, out_hbm, *s):
    tidx = s[0:2]
    cidx = s[2:4]
    thv = s[4:6]
    chv = s[6:8]
    tbv = s[8:10]
    cbv = s[10:12]
    tgt_v = s[12:14]
    ctx_v = s[14:16]
    out_v = s[16:18]
    semi = s[18:20]
    semg = s[20:22]
    semo = s[22:24]
    wid = lax.axis_index("s") * _NC + lax.axis_index("c")
    iota = lax.broadcasted_iota(jnp.int32, (16,), 0)
    iota_ctx = iota * CTX

    def prep_idx(g, b):
        b0 = wid * _BPW + g * _GB
        pltpu.async_copy(t_hbm.at[pl.ds(b0, _GB)], tidx[b], semi[b])
        pltpu.async_copy(c_hbm.at[pl.ds(b0 * CTX, _ROWS)], cidx[b], semi[b])

    def wait_idx(b):
        pltpu.make_async_copy(t_hbm.at[pl.ds(0, _GB)], tidx[b], semi[b]).wait()
        pltpu.make_async_copy(c_hbm.at[pl.ds(0, _ROWS)], cidx[b],
                              semi[b]).wait()

    def launch_gather(b):
        tid = tidx[b][...]
        thv[b][...] = ((tid >> _VSH) << _QSH) | (tid & _QM)
        tbv[b][...] = ((tid >> _QSH) & 3) << 5
        for k in range(CTX):
            cv = cidx[b][pl.ds(k * _GB, _GB)]
            chv[b][pl.ds(k * _GB, _GB)] = ((cv >> _VSH) << _QSH) | (cv & _QM)
            cbv[b][pl.ds(k * _GB, _GB)] = ((cv >> _QSH) & 3) << 5
        pltpu.async_copy(skip_hbm.at[thv[b]], tgt_v[b], semg[b])
        pltpu.async_copy(ctxtab_hbm.at[chv[b]], ctx_v[b], semg[b])

    def wait_gather(b):
        pltpu.make_async_copy(skip_hbm.at[pl.ds(0, _GB)], tgt_v[b],
                              semg[b]).wait()
        pltpu.make_async_copy(skip_hbm.at[pl.ds(0, _ROWS)], ctx_v[b],
                              semg[b]).wait()

    def compute_out(g, b, t2):
        tb32 = tbv[b][...]
        outs = []
        for h in range(2):
            cs = range(h * _HALF, (h + 1) * _HALF)
            cb32 = [plsc.load_gather(cbv[b], [iota_ctx + c]) for c in cs]

            def wstep(w, accs):
                tw = plsc.load_gather(tgt_v[b], [iota, tb32 + w])
                ta, tb = plsc.unpack(plsc.bitcast(tw, jnp.bfloat16),
                                     format=plsc.PackFormat.INTERLEAVED)
                nxt = []
                for j, c in enumerate(cs):
                    cw = plsc.load_gather(ctx_v[b],
                                          [iota_ctx + c, cb32[j] + w])
                    ca, cb = plsc.unpack(plsc.bitcast(cw, jnp.bfloat16),
                                         format=plsc.PackFormat.INTERLEAVED)
                    nxt.append(accs[j] + ta * ca + tb * cb)
                return tuple(nxt)

            outs.append(lax.fori_loop(
                0, WPR, wstep,
                tuple(jnp.zeros((16,), jnp.float32) for _ in cs)))

        @pl.when(t2 > 0)
        def _():
            pltpu.make_async_copy(out_v[b], out_hbm.at[pl.ds(0, _ROWS)],
                                  semo[b]).wait()

        for h in range(2):
            for j, c in enumerate(range(h * _HALF, (h + 1) * _HALF)):
                plsc.store_scatter(out_v[b], [iota_ctx + c], outs[h][j])
        p0 = (wid * _BPW + g * _GB) * CTX
        pltpu.async_copy(out_v[b], out_hbm.at[pl.ds(p0, _ROWS)], semo[b])

    prep_idx(0, 0)
    wait_idx(0)
    launch_gather(0)
    prep_idx(1, 1)

    def body2(t2, carry):
        g0 = 2 * t2
        wait_idx(1)
        launch_gather(1)

        @pl.when(g0 + 2 < _NG)
        def _():
            prep_idx(g0 + 2, 0)

        wait_gather(0)
        compute_out(g0, 0, t2)

        @pl.when(g0 + 2 < _NG)
        def _():
            wait_idx(0)
            launch_gather(0)

        @pl.when(g0 + 3 < _NG)
        def _():
            prep_idx(g0 + 3, 1)

        wait_gather(1)
        compute_out(g0 + 1, 1, t2)
        return carry

    lax.fori_loop(0, _NG // 2, body2, 0)
    for b in range(2):
        pltpu.make_async_copy(out_v[b], out_hbm.at[pl.ds(0, _ROWS)],
                              semo[b]).wait()


def _pack_body(tt_ref, out_ref):
    xb = tt_ref[...].astype(jnp.bfloat16)
    r = lax.broadcasted_iota(jnp.int32, (DIM, WPR), 0)
    c = lax.broadcasted_iota(jnp.int32, (DIM, WPR), 1)
    ev = (r == 2 * c).astype(jnp.bfloat16)
    od = (r == 2 * c + 1).astype(jnp.bfloat16)
    dn = (((0,), (0,)), ((), ()))
    lo = lax.dot_general(xb, ev, dn, preferred_element_type=jnp.float32)
    hi = lax.dot_general(xb, od, dn, preferred_element_type=jnp.float32)
    li = jax.lax.bitcast_convert_type(lo, jnp.int32)
    hj = jax.lax.bitcast_convert_type(hi, jnp.int32)
    w = (hj & jnp.int32(-65536)) | lax.shift_right_logical(li, 16)
    q = _VCH // 4
    out_ref[...] = jnp.concatenate(
        [w[a * q:(a + 1) * q, :] for a in range(4)], axis=1)


def _pack_table(tab):
    f = pl.pallas_call(
        _pack_body,
        grid=(_TGRID,),
        in_specs=[pl.BlockSpec((DIM, _VCH), lambda k: (0, k))],
        out_specs=pl.BlockSpec((_VCH // 4, PACK), lambda k: (k, 0)),
        out_shape=jax.ShapeDtypeStruct((_TGRID * (_VCH // 4), PACK), jnp.int32),
    )
    return f(tab.T)


def kernel(target, context, skipgram_table, context_table):
    mesh = plsc.VectorSubcoreMesh(core_axis_name="c", subcore_axis_name="s")
    f = pl.kernel(
        _sc_body,
        out_type=jax.ShapeDtypeStruct((BATCH * CTX,), jnp.float32),
        mesh=mesh,
        scratch_types=(
            [pltpu.VMEM((_GB,), jnp.int32)] * 2
            + [pltpu.VMEM((_ROWS,), jnp.int32)] * 2
            + [pltpu.VMEM((_GB,), jnp.int32)] * 2
            + [pltpu.VMEM((_ROWS,), jnp.int32)] * 2
            + [pltpu.VMEM((_GB,), jnp.int32)] * 2
            + [pltpu.VMEM((_ROWS,), jnp.int32)] * 2
            + [pltpu.VMEM((_GB, PACK), jnp.int32)] * 2
            + [pltpu.VMEM((_ROWS, PACK), jnp.int32)] * 2
            + [pltpu.VMEM((_ROWS,), jnp.float32)] * 2
            + [pltpu.SemaphoreType.DMA] * 6
        ),
        compiler_params=pltpu.CompilerParams(
            needs_layout_passes=False, use_tc_tiling_on_sc=True),
    )
    out = f(target.astype(jnp.int32), context.reshape(-1).astype(jnp.int32),
            _pack_table(skipgram_table), _pack_table(context_table))
    return out.reshape(BATCH, CTX)
